# baseline (speedup 1.0000x reference)
import functools
import jax
import jax.numpy as jnp
from jax import lax
from jax.experimental import pallas as pl
from jax.experimental.pallas import tpu as pltpu
from jax.experimental.pallas import tpu_sc as plsc

NC = 2
NS = 16
NW = NC * NS
L = 16


def _gather_positional_rows(pos_embedding, T):
    V, D = pos_embedding.shape
    WT = T // NW
    mesh = plsc.VectorSubcoreMesh(
        core_axis_name="c", subcore_axis_name="s",
        num_cores=NC, num_subcores=NS,
    )

    @functools.partial(
        pl.kernel,
        out_type=jax.ShapeDtypeStruct((T, D), jnp.float32),
        mesh=mesh,
        scratch_types=[
            pltpu.VMEM((WT,), jnp.int32),
            pltpu.VMEM((WT, D), jnp.float32),
            pltpu.SemaphoreType.DMA,
        ],
    )
    def sc_gather(pe_hbm, out_hbm, idx_v, rows_v, sem):
        wid = lax.axis_index("s") * NC + lax.axis_index("c")
        t_base = wid * WT
        for j in range(WT // L):
            idx_v[pl.ds(j * L, L)] = t_base + j * L + lax.iota(jnp.int32, L)
        pltpu.async_copy(pe_hbm.at[idx_v], rows_v, sem).wait()
        pltpu.sync_copy(rows_v, out_hbm.at[pl.ds(t_base, WT)])

    return sc_gather(pos_embedding)


def _tc_add_body(S):
    def body(x_ref, pe_ref, out_ref):
        pe = pe_ref[...]
        for s in range(S):
            out_ref[0, :, s, :] = x_ref[0, :, s, :] + pe
    return body


def _tc_add(x, rows):
    B, T, S, D = x.shape
    TB = 512
    grid = (T // TB, B)
    return pl.pallas_call(
        _tc_add_body(S),
        grid=grid,
        in_specs=[
            pl.BlockSpec((1, TB, S, D), lambda t, b: (b, t, 0, 0)),
            pl.BlockSpec((TB, D), lambda t, b: (t, 0)),
        ],
        out_specs=pl.BlockSpec((1, TB, S, D), lambda t, b: (b, t, 0, 0)),
        out_shape=jax.ShapeDtypeStruct((B, T, S, D), x.dtype),
    )(x, rows)


def kernel(x, pos_embedding):
    B, T, S, D = x.shape
    rows = _gather_positional_rows(pos_embedding, T)
    return _tc_add(x, rows)
